# baseline (speedup 1.0000x reference)
import functools

import jax
import jax.numpy as jnp
from jax import lax
from jax.experimental import pallas as pl
from jax.experimental.pallas import tpu as pltpu
from jax.experimental.pallas import tpu_sc as plsc

N = 10000
E = 320000
D = 128
NC = 2
NS = 16
NW = NC * NS
CHUNK = 128
EPW = 10240
NCHUNK = EPW // CHUNK
GRP = 8
NGRP = NCHUNK // GRP
E_PAD = NW * EPW
ACC_ROWS = 10240
RPT = ACC_ROWS // NS
DUMMY = N
HROWS = ACC_ROWS // 16

_mesh = plsc.VectorSubcoreMesh(core_axis_name="c", subcore_axis_name="s")
_sc_params = pltpu.CompilerParams(needs_layout_passes=False)


@functools.partial(
    pl.kernel,
    out_type=jax.ShapeDtypeStruct((NC, HROWS, 16), jnp.float32),
    mesh=_mesh,
    compiler_params=_sc_params,
    scratch_types=[
        pltpu.VMEM((NCHUNK, CHUNK), jnp.int32),
        pltpu.VMEM((HROWS, 16), jnp.float32),
        pltpu.VMEM((HROWS // NS, 16), jnp.float32),
        pltpu.VMEM((HROWS // CHUNK, CHUNK), jnp.int32),
        pltpu.VMEM_SHARED((HROWS, 16), jnp.float32),
    ],
)
def _sc_degree(dst_hbm, deg_hbm, dstv, hist, zbuf, rowidx, deg_sh):
  cid = lax.axis_index("c")
  sid = lax.axis_index("s")
  wid = sid * NC + cid
  zero16 = jnp.zeros((16,), jnp.float32)
  rpt = HROWS // NS

  pltpu.sync_copy(dst_hbm.at[wid], dstv)
  for r in range(rpt):
    zbuf[r, :] = zero16
  pltpu.sync_copy(zbuf, deg_sh.at[pl.ds(sid * rpt, rpt), :])

  def _zero_hist(i, c):
    hist[i, :] = zero16
    return c
  lax.fori_loop(0, HROWS, _zero_hist, 0)

  for c in range(HROWS // CHUNK):
    for k in range(CHUNK // 16):
      rowidx[c, pl.ds(k * 16, 16)] = (
          c * CHUNK + k * 16 + lax.iota(jnp.int32, 16))
  plsc.subcore_barrier()

  one16 = jnp.ones((16,), jnp.float32)

  def _accum(j, c):
    for k in range(CHUNK // 16):
      d = dstv[j, pl.ds(k * 16, 16)]
      plsc.addupdate_scatter(hist, [d >> 4, d & 15], one16)
    return c
  lax.fori_loop(0, NCHUNK, _accum, 0)

  for c in range(HROWS // CHUNK):
    pltpu.sync_copy(hist.at[pl.ds(c * CHUNK, CHUNK), :],
                    deg_sh.at[rowidx.at[c]], add=True)
  plsc.subcore_barrier()

  pltpu.sync_copy(deg_sh.at[pl.ds(sid * rpt, rpt), :], zbuf)
  pltpu.sync_copy(zbuf, deg_hbm.at[cid].at[pl.ds(sid * rpt, rpt), :])


@functools.partial(
    pl.kernel,
    out_type=jax.ShapeDtypeStruct((NC, ACC_ROWS, D), jnp.float32),
    mesh=_mesh,
    compiler_params=_sc_params,
    scratch_types=[
        pltpu.VMEM((GRP, CHUNK), jnp.int32),
        pltpu.VMEM((GRP, CHUNK), jnp.int32),
        pltpu.VMEM((NCHUNK, CHUNK), jnp.int32),
        pltpu.VMEM((CHUNK, D), jnp.float32),
        pltpu.VMEM((CHUNK, D), jnp.float32),
        pltpu.VMEM_SHARED((ACC_ROWS, D), jnp.float32),
        pltpu.SemaphoreType.DMA,
        pltpu.SemaphoreType.DMA,
        pltpu.SemaphoreType.DMA,
    ],
)
def _sc_scatter(g_hbm, src_hbm, dst_hbm, out_hbm,
                srcA, srcB, dstv, buf0, buf1, acc, gsem0, gsem1, psem):
  cid = lax.axis_index("c")
  sid = lax.axis_index("s")
  wid = sid * NC + cid
  zero16 = jnp.zeros((16,), jnp.float32)

  bufs = (buf0, buf1)
  gsems = (gsem0, gsem1)
  sbufs = (srcA, srcB)

  def src_row(jj):
    return sbufs[(jj // GRP) % 2].at[jj % GRP]

  pltpu.sync_copy(dst_hbm.at[wid], dstv)
  pltpu.sync_copy(src_hbm.at[wid].at[pl.ds(0, GRP)], srcA)
  pref = pltpu.async_copy(src_hbm.at[wid].at[pl.ds(GRP, GRP)], srcB, psem)

  def _zero(i, c):
    for k in range(D // 16):
      buf0[i, pl.ds(k * 16, 16)] = zero16
    return c
  lax.fori_loop(0, CHUNK, _zero, 0)
  for c in range(RPT // CHUNK):
    pltpu.sync_copy(buf0, acc.at[pl.ds(sid * RPT + c * CHUNK, CHUNK), :])

  pend = [None, None]
  pend[0] = pltpu.async_copy(g_hbm.at[src_row(0)], buf0, gsem0)
  plsc.subcore_barrier()

  for j in range(NCHUNK):
    b = j % 2
    g = j // GRP
    if j % GRP == 0 and 0 < g and g + 1 < NGRP:
      pref = pltpu.async_copy(
          src_hbm.at[wid].at[pl.ds((g + 1) * GRP, GRP)], sbufs[(g + 1) % 2],
          psem)
    pend[b].wait()
    jn = j + 1
    if jn < NCHUNK:
      if jn % GRP == 0:
        pref.wait()
      pend[1 - b] = pltpu.async_copy(
          g_hbm.at[src_row(jn)], bufs[1 - b], gsems[1 - b])
    pltpu.sync_copy(bufs[b], acc.at[dstv.at[j]], add=True)
  plsc.subcore_barrier()

  for c in range(RPT // CHUNK):
    pltpu.sync_copy(acc.at[pl.ds(sid * RPT + c * CHUNK, CHUNK), :], buf0)
    pltpu.sync_copy(
        buf0, out_hbm.at[cid].at[pl.ds(sid * RPT + c * CHUNK, CHUNK), :])


_BLK = 1000


def _tc_pre_body(x_ref, w_ref, da_ref, db_ref, g_ref):
  deg = da_ref[...] + db_ref[...]
  dis = lax.rsqrt(deg)
  g_ref[...] = jnp.dot(x_ref[...], w_ref[...],
                       preferred_element_type=jnp.float32) * dis


def _tc_mid_body(a_ref, b_ref, da_ref, db_ref, w_ref, g_ref):
  deg = da_ref[...] + db_ref[...]
  dis = lax.rsqrt(deg)
  dis_safe = jnp.where(deg > 0, dis, 0.0)
  h = jnp.maximum((a_ref[...] + b_ref[...]) * dis_safe, 0.0)
  g_ref[...] = jnp.dot(h, w_ref[...],
                       preferred_element_type=jnp.float32) * dis


def _tc_post_body(a_ref, b_ref, da_ref, db_ref, o_ref):
  deg = da_ref[...] + db_ref[...]
  dis_safe = jnp.where(deg > 0, lax.rsqrt(deg), 0.0)
  o_ref[...] = jnp.maximum((a_ref[...] + b_ref[...]) * dis_safe, 0.0)


_row_spec = pl.BlockSpec((_BLK, D), lambda i: (i, 0))
_deg_spec = pl.BlockSpec((_BLK, 1), lambda i: (i, 0))
_w_spec = pl.BlockSpec((D, D), lambda i: (0, 0))
_out_struct = jax.ShapeDtypeStruct((N, D), jnp.float32)

_tc_pre = pl.pallas_call(
    _tc_pre_body,
    grid=(N // _BLK,),
    in_specs=[_row_spec, _w_spec, _deg_spec, _deg_spec],
    out_specs=_row_spec,
    out_shape=_out_struct,
)

_tc_mid = pl.pallas_call(
    _tc_mid_body,
    grid=(N // _BLK,),
    in_specs=[_row_spec, _row_spec, _deg_spec, _deg_spec, _w_spec],
    out_specs=_row_spec,
    out_shape=_out_struct,
)

_tc_post = pl.pallas_call(
    _tc_post_body,
    grid=(N // _BLK,),
    in_specs=[_row_spec, _row_spec, _deg_spec, _deg_spec],
    out_specs=_row_spec,
    out_shape=_out_struct,
)


REAL_PW = E // NW
PAD_PW = EPW - REAL_PW


@jax.jit
def kernel(x, edge_index, W1, W2):
  ei = edge_index.astype(jnp.int32)
  woff = (jnp.arange(NW, dtype=jnp.int32)[:, None] // NC) * (PAD_PW // NS)
  pad_dst = N + (woff + jnp.arange(PAD_PW, dtype=jnp.int32)[None, :]) % PAD_PW
  src = jnp.concatenate(
      [ei[0].reshape(NW, REAL_PW), jnp.zeros((NW, PAD_PW), jnp.int32)],
      axis=1).reshape(NW, NCHUNK, CHUNK)
  dst = jnp.concatenate(
      [ei[1].reshape(NW, REAL_PW), pad_dst],
      axis=1).reshape(NW, NCHUNK, CHUNK)

  deg_parts = _sc_degree(dst)
  deg_a = deg_parts[0].reshape(ACC_ROWS)[:N].reshape(N, 1)
  deg_b = deg_parts[1].reshape(ACC_ROWS)[:N].reshape(N, 1)

  g1 = _tc_pre(x, W1, deg_a, deg_b)
  acc1 = _sc_scatter(g1, src, dst)
  g2 = _tc_mid(acc1[0, :N], acc1[1, :N], deg_a, deg_b, W2)
  acc2 = _sc_scatter(g2, src, dst)
  return _tc_post(acc2[0, :N], acc2[1, :N], deg_a, deg_b)


# speedup vs baseline: 2.9184x; 2.9184x over previous
import functools

import jax
import jax.numpy as jnp
from jax import lax
from jax.experimental import pallas as pl
from jax.experimental.pallas import tpu as pltpu
from jax.experimental.pallas import tpu_sc as plsc

N = 10000
E = 320000
D = 128
NC = 2
NS = 16
NW = NC * NS
CHUNK = 128
EPW = 10240
NCHUNK = EPW // CHUNK
GRP = 8
NGRP = NCHUNK // GRP
E_PAD = NW * EPW
ACC_ROWS = 10240
RPT = ACC_ROWS // NS
DUMMY = N
HROWS = ACC_ROWS // 16

_mesh = plsc.VectorSubcoreMesh(core_axis_name="c", subcore_axis_name="s")
_sc_params = pltpu.CompilerParams(needs_layout_passes=False)


@functools.partial(
    pl.kernel,
    out_type=jax.ShapeDtypeStruct((NC, HROWS, 16), jnp.float32),
    mesh=_mesh,
    compiler_params=_sc_params,
    scratch_types=[
        pltpu.VMEM((NCHUNK, CHUNK), jnp.int32),
        pltpu.VMEM((HROWS, 16), jnp.float32),
        pltpu.VMEM((HROWS // NS, 16), jnp.float32),
        pltpu.VMEM((HROWS // CHUNK, CHUNK), jnp.int32),
        pltpu.VMEM_SHARED((HROWS, 16), jnp.float32),
    ],
)
def _sc_degree(dst_hbm, deg_hbm, dstv, hist, zbuf, rowidx, deg_sh):
  cid = lax.axis_index("c")
  sid = lax.axis_index("s")
  wid = sid * NC + cid
  zero16 = jnp.zeros((16,), jnp.float32)
  rpt = HROWS // NS

  pltpu.sync_copy(dst_hbm.at[wid], dstv)
  for r in range(rpt):
    zbuf[r, :] = zero16
  pltpu.sync_copy(zbuf, deg_sh.at[pl.ds(sid * rpt, rpt), :])

  def _zero_hist(i, c):
    hist[i, :] = zero16
    return c
  lax.fori_loop(0, HROWS, _zero_hist, 0)

  for c in range(HROWS // CHUNK):
    for k in range(CHUNK // 16):
      rowidx[c, pl.ds(k * 16, 16)] = (
          c * CHUNK + k * 16 + lax.iota(jnp.int32, 16))
  plsc.subcore_barrier()

  one16 = jnp.ones((16,), jnp.float32)

  def _accum(j, c):
    for k in range(CHUNK // 16):
      d = dstv[j, pl.ds(k * 16, 16)]
      plsc.addupdate_scatter(hist, [d >> 4, d & 15], one16)
    return c
  lax.fori_loop(0, NCHUNK, _accum, 0)

  for c in range(HROWS // CHUNK):
    pltpu.sync_copy(hist.at[pl.ds(c * CHUNK, CHUNK), :],
                    deg_sh.at[rowidx.at[c]], add=True)
  plsc.subcore_barrier()

  pltpu.sync_copy(deg_sh.at[pl.ds(sid * rpt, rpt), :], zbuf)
  pltpu.sync_copy(zbuf, deg_hbm.at[cid].at[pl.ds(sid * rpt, rpt), :])


@functools.partial(
    pl.kernel,
    out_type=jax.ShapeDtypeStruct((NC, ACC_ROWS, D), jnp.float32),
    mesh=_mesh,
    compiler_params=_sc_params,
    scratch_types=[
        pltpu.VMEM((GRP, CHUNK), jnp.int32),
        pltpu.VMEM((GRP, CHUNK), jnp.int32),
        pltpu.VMEM((NCHUNK, CHUNK), jnp.int32),
        pltpu.VMEM((CHUNK, D), jnp.float32),
        pltpu.VMEM((CHUNK, D), jnp.float32),
        pltpu.VMEM_SHARED((ACC_ROWS, D), jnp.float32),
        pltpu.SemaphoreType.DMA,
        pltpu.SemaphoreType.DMA,
        pltpu.SemaphoreType.DMA,
    ],
)
def _sc_scatter(g_hbm, src_hbm, dst_hbm, out_hbm,
                srcA, srcB, dstv, buf0, buf1, acc, gsem0, gsem1, psem):
  cid = lax.axis_index("c")
  sid = lax.axis_index("s")
  wid = sid * NC + cid
  zero16 = jnp.zeros((16,), jnp.float32)

  bufs = (buf0, buf1)
  gsems = (gsem0, gsem1)
  sbufs = (srcA, srcB)

  def src_row(jj):
    return sbufs[(jj // GRP) % 2].at[jj % GRP]

  pltpu.sync_copy(dst_hbm.at[wid], dstv)
  pltpu.sync_copy(src_hbm.at[wid].at[pl.ds(0, GRP)], srcA)
  pref = pltpu.async_copy(src_hbm.at[wid].at[pl.ds(GRP, GRP)], srcB, psem)

  def _zero(i, c):
    for k in range(D // 16):
      buf0[i, pl.ds(k * 16, 16)] = zero16
    return c
  lax.fori_loop(0, CHUNK, _zero, 0)
  for c in range(RPT // CHUNK):
    pltpu.sync_copy(buf0, acc.at[pl.ds(sid * RPT + c * CHUNK, CHUNK), :])

  pend = [None, None]
  pend[0] = pltpu.async_copy(g_hbm.at[src_row(0)], buf0, gsem0)
  plsc.subcore_barrier()

  for j in range(NCHUNK):
    b = j % 2
    g = j // GRP
    if j % GRP == 0 and 0 < g and g + 1 < NGRP:
      pref = pltpu.async_copy(
          src_hbm.at[wid].at[pl.ds((g + 1) * GRP, GRP)], sbufs[(g + 1) % 2],
          psem)
    jn = j + 1
    if jn < NCHUNK:
      if jn % GRP == 0:
        pref.wait()
      pend[1 - b] = pltpu.async_copy(
          g_hbm.at[src_row(jn)], bufs[1 - b], gsems[1 - b])
    pend[b].wait()
    pltpu.sync_copy(bufs[b], acc.at[dstv.at[j]], add=True)
  plsc.subcore_barrier()

  for c in range(RPT // CHUNK):
    pltpu.sync_copy(acc.at[pl.ds(sid * RPT + c * CHUNK, CHUNK), :], buf0)
    pltpu.sync_copy(
        buf0, out_hbm.at[cid].at[pl.ds(sid * RPT + c * CHUNK, CHUNK), :])


_BLK = 1000


def _tc_pre_body(x_ref, w_ref, da_ref, db_ref, g_ref):
  deg = da_ref[...] + db_ref[...]
  dis = lax.rsqrt(deg)
  g_ref[...] = jnp.dot(x_ref[...], w_ref[...],
                       preferred_element_type=jnp.float32) * dis


def _tc_mid_body(a_ref, b_ref, da_ref, db_ref, w_ref, g_ref):
  deg = da_ref[...] + db_ref[...]
  dis = lax.rsqrt(deg)
  dis_safe = jnp.where(deg > 0, dis, 0.0)
  h = jnp.maximum((a_ref[...] + b_ref[...]) * dis_safe, 0.0)
  g_ref[...] = jnp.dot(h, w_ref[...],
                       preferred_element_type=jnp.float32) * dis


def _tc_post_body(a_ref, b_ref, da_ref, db_ref, o_ref):
  deg = da_ref[...] + db_ref[...]
  dis_safe = jnp.where(deg > 0, lax.rsqrt(deg), 0.0)
  o_ref[...] = jnp.maximum((a_ref[...] + b_ref[...]) * dis_safe, 0.0)


_row_spec = pl.BlockSpec((_BLK, D), lambda i: (i, 0))
_deg_spec = pl.BlockSpec((_BLK, 1), lambda i: (i, 0))
_w_spec = pl.BlockSpec((D, D), lambda i: (0, 0))
_out_struct = jax.ShapeDtypeStruct((N, D), jnp.float32)

_tc_pre = pl.pallas_call(
    _tc_pre_body,
    grid=(N // _BLK,),
    in_specs=[_row_spec, _w_spec, _deg_spec, _deg_spec],
    out_specs=_row_spec,
    out_shape=_out_struct,
)

_tc_mid = pl.pallas_call(
    _tc_mid_body,
    grid=(N // _BLK,),
    in_specs=[_row_spec, _row_spec, _deg_spec, _deg_spec, _w_spec],
    out_specs=_row_spec,
    out_shape=_out_struct,
)

_tc_post = pl.pallas_call(
    _tc_post_body,
    grid=(N // _BLK,),
    in_specs=[_row_spec, _row_spec, _deg_spec, _deg_spec],
    out_specs=_row_spec,
    out_shape=_out_struct,
)


REAL_PW = E // NW
PAD_PW = EPW - REAL_PW


@jax.jit
def kernel(x, edge_index, W1, W2):
  ei = edge_index.astype(jnp.int32)
  woff = (jnp.arange(NW, dtype=jnp.int32)[:, None] // NC) * (PAD_PW // NS)
  pad_dst = N + (woff + jnp.arange(PAD_PW, dtype=jnp.int32)[None, :]) % PAD_PW
  pad_src = (jnp.arange(NW, dtype=jnp.int32)[:, None] * PAD_PW
             + jnp.arange(PAD_PW, dtype=jnp.int32)[None, :])
  src = jnp.concatenate(
      [ei[0].reshape(NW, REAL_PW), pad_src],
      axis=1).reshape(NW, NCHUNK, CHUNK)
  dst = jnp.concatenate(
      [ei[1].reshape(NW, REAL_PW), pad_dst],
      axis=1).reshape(NW, NCHUNK, CHUNK)

  deg_parts = _sc_degree(dst)
  deg_a = deg_parts[0].reshape(ACC_ROWS)[:N].reshape(N, 1)
  deg_b = deg_parts[1].reshape(ACC_ROWS)[:N].reshape(N, 1)

  g1 = _tc_pre(x, W1, deg_a, deg_b)
  acc1 = _sc_scatter(g1, src, dst)
  g2 = _tc_mid(acc1[0, :N], acc1[1, :N], deg_a, deg_b, W2)
  acc2 = _sc_scatter(g2, src, dst)
  return _tc_post(acc2[0, :N], acc2[1, :N], deg_a, deg_b)


# speedup vs baseline: 2.9257x; 1.0025x over previous
import functools

import jax
import jax.numpy as jnp
from jax import lax
from jax.experimental import pallas as pl
from jax.experimental.pallas import tpu as pltpu
from jax.experimental.pallas import tpu_sc as plsc

N = 10000
E = 320000
D = 128
NC = 2
NS = 16
NW = NC * NS
CHUNK = 128
EPW = 10240
NCHUNK = EPW // CHUNK
GRP = 8
NGRP = NCHUNK // GRP
E_PAD = NW * EPW
ACC_ROWS = 10240
RPT = ACC_ROWS // NS
HROWS = ACC_ROWS // 16

_mesh = plsc.VectorSubcoreMesh(core_axis_name="c", subcore_axis_name="s")
_sc_params = pltpu.CompilerParams(needs_layout_passes=False)


@functools.partial(
    pl.kernel,
    out_type=jax.ShapeDtypeStruct((NC, HROWS, 16), jnp.float32),
    mesh=_mesh,
    compiler_params=_sc_params,
    scratch_types=[
        pltpu.VMEM((NCHUNK, CHUNK), jnp.int32),
        pltpu.VMEM((HROWS, 16), jnp.float32),
        pltpu.VMEM((HROWS // NS, 16), jnp.float32),
        pltpu.VMEM((HROWS // CHUNK, CHUNK), jnp.int32),
        pltpu.VMEM_SHARED((HROWS, 16), jnp.float32),
    ],
)
def _sc_degree(dst_hbm, deg_hbm, dstv, hist, zbuf, rowidx, deg_sh):
  cid = lax.axis_index("c")
  sid = lax.axis_index("s")
  wid = sid * NC + cid
  zero16 = jnp.zeros((16,), jnp.float32)
  rpt = HROWS // NS

  pltpu.sync_copy(dst_hbm.at[wid], dstv)
  for r in range(rpt):
    zbuf[r, :] = zero16
  pltpu.sync_copy(zbuf, deg_sh.at[pl.ds(sid * rpt, rpt), :])

  def _zero_hist(i, c):
    hist[i, :] = zero16
    return c
  lax.fori_loop(0, HROWS, _zero_hist, 0)

  for c in range(HROWS // CHUNK):
    for k in range(CHUNK // 16):
      rowidx[c, pl.ds(k * 16, 16)] = (
          c * CHUNK + k * 16 + lax.iota(jnp.int32, 16))
  plsc.subcore_barrier()

  one16 = jnp.ones((16,), jnp.float32)

  def _accum(j, c):
    for k in range(CHUNK // 16):
      d = dstv[j, pl.ds(k * 16, 16)]
      plsc.addupdate_scatter(hist, [d >> 4, d & 15], one16)
    return c
  lax.fori_loop(0, NCHUNK, _accum, 0)

  for c in range(HROWS // CHUNK):
    pltpu.sync_copy(hist.at[pl.ds(c * CHUNK, CHUNK), :],
                    deg_sh.at[rowidx.at[c]], add=True)
  plsc.subcore_barrier()

  pltpu.sync_copy(deg_sh.at[pl.ds(sid * rpt, rpt), :], zbuf)
  pltpu.sync_copy(zbuf, deg_hbm.at[cid].at[pl.ds(sid * rpt, rpt), :])


@functools.partial(
    pl.kernel,
    out_type=jax.ShapeDtypeStruct((NC, ACC_ROWS, D), jnp.float32),
    mesh=_mesh,
    compiler_params=_sc_params,
    scratch_types=[
        pltpu.VMEM((GRP, CHUNK), jnp.int32),
        pltpu.VMEM((GRP, CHUNK), jnp.int32),
        pltpu.VMEM((NCHUNK, CHUNK), jnp.int32),
        pltpu.VMEM((CHUNK, D), jnp.float32),
        pltpu.VMEM((CHUNK, D), jnp.float32),
        pltpu.VMEM_SHARED((ACC_ROWS, D), jnp.float32),
        pltpu.SemaphoreType.DMA,
        pltpu.SemaphoreType.DMA,
        pltpu.SemaphoreType.DMA,
    ],
)
def _sc_scatter(g_hbm, src_hbm, dst_hbm, out_hbm,
                srcA, srcB, dstv, buf0, buf1, acc, gsem0, gsem1, psem):
  cid = lax.axis_index("c")
  sid = lax.axis_index("s")
  wid = sid * NC + cid
  zero16 = jnp.zeros((16,), jnp.float32)

  bufs = (buf0, buf1)
  gsems = (gsem0, gsem1)
  sbufs = (srcA, srcB)

  def src_row(jj):
    return sbufs[(jj // GRP) % 2].at[jj % GRP]

  pltpu.sync_copy(dst_hbm.at[wid], dstv)
  pltpu.sync_copy(src_hbm.at[wid].at[pl.ds(0, GRP)], srcA)
  pref = pltpu.async_copy(src_hbm.at[wid].at[pl.ds(GRP, GRP)], srcB, psem)

  def _zero(i, c):
    for k in range(D // 16):
      buf0[i, pl.ds(k * 16, 16)] = zero16
    return c
  lax.fori_loop(0, CHUNK, _zero, 0)
  for c in range(RPT // CHUNK):
    pltpu.sync_copy(buf0, acc.at[pl.ds(sid * RPT + c * CHUNK, CHUNK), :])

  pend = [None, None]
  pend[0] = pltpu.async_copy(g_hbm.at[src_row(0)], buf0, gsem0)
  plsc.subcore_barrier()

  for j in range(NCHUNK):
    b = j % 2
    g = j // GRP
    if j % GRP == 0 and 0 < g and g + 1 < NGRP:
      pref = pltpu.async_copy(
          src_hbm.at[wid].at[pl.ds((g + 1) * GRP, GRP)], sbufs[(g + 1) % 2],
          psem)
    jn = j + 1
    if jn < NCHUNK:
      if jn % GRP == 0:
        pref.wait()
      pend[1 - b] = pltpu.async_copy(
          g_hbm.at[src_row(jn)], bufs[1 - b], gsems[1 - b])
    pend[b].wait()
    pltpu.sync_copy(bufs[b], acc.at[dstv.at[j]], add=True)
  plsc.subcore_barrier()

  for c in range(RPT // CHUNK):
    pltpu.sync_copy(acc.at[pl.ds(sid * RPT + c * CHUNK, CHUNK), :], buf0)
    pltpu.sync_copy(
        buf0, out_hbm.at[cid].at[pl.ds(sid * RPT + c * CHUNK, CHUNK), :])


_BLK = 1000


def _tc_pre_body(x_ref, w_ref, da_ref, db_ref, g_ref):
  deg = da_ref[...] + db_ref[...]
  dis = lax.rsqrt(deg)
  g_ref[...] = jnp.dot(x_ref[...], w_ref[...],
                       preferred_element_type=jnp.float32) * dis


def _tc_mid_body(a_ref, b_ref, da_ref, db_ref, w_ref, g_ref):
  deg = da_ref[...] + db_ref[...]
  dis = lax.rsqrt(deg)
  dis_safe = jnp.where(deg > 0, dis, 0.0)
  h = jnp.maximum((a_ref[...] + b_ref[...]) * dis_safe, 0.0)
  g_ref[...] = jnp.dot(h, w_ref[...],
                       preferred_element_type=jnp.float32) * dis


def _tc_post_body(a_ref, b_ref, da_ref, db_ref, o_ref):
  deg = da_ref[...] + db_ref[...]
  dis_safe = jnp.where(deg > 0, lax.rsqrt(deg), 0.0)
  o_ref[...] = jnp.maximum((a_ref[...] + b_ref[...]) * dis_safe, 0.0)


_row_spec = pl.BlockSpec((_BLK, D), lambda i: (i, 0))
_deg_spec = pl.BlockSpec((_BLK, 1), lambda i: (i, 0))
_w_spec = pl.BlockSpec((D, D), lambda i: (0, 0))
_out_struct = jax.ShapeDtypeStruct((N, D), jnp.float32)

_tc_pre = pl.pallas_call(
    _tc_pre_body,
    grid=(N // _BLK,),
    in_specs=[_row_spec, _w_spec, _deg_spec, _deg_spec],
    out_specs=_row_spec,
    out_shape=_out_struct,
)

_tc_mid = pl.pallas_call(
    _tc_mid_body,
    grid=(N // _BLK,),
    in_specs=[_row_spec, _row_spec, _deg_spec, _deg_spec, _w_spec],
    out_specs=_row_spec,
    out_shape=_out_struct,
)

_tc_post = pl.pallas_call(
    _tc_post_body,
    grid=(N // _BLK,),
    in_specs=[_row_spec, _row_spec, _deg_spec, _deg_spec],
    out_specs=_row_spec,
    out_shape=_out_struct,
)


_PAD_SRC = jnp.arange(E_PAD - E, dtype=jnp.int32) % N
_PAD_DST = N + jnp.arange(E_PAD - E, dtype=jnp.int32) % (ACC_ROWS - N)


@jax.jit
def kernel(x, edge_index, W1, W2):
  ei = edge_index.astype(jnp.int32)
  src = jnp.concatenate([ei[0], _PAD_SRC]).reshape(NW, NCHUNK, CHUNK)
  dst = jnp.concatenate([ei[1], _PAD_DST]).reshape(NW, NCHUNK, CHUNK)

  deg_parts = _sc_degree(dst)
  deg_a = deg_parts[0].reshape(ACC_ROWS, 1)
  deg_b = deg_parts[1].reshape(ACC_ROWS, 1)

  g1 = _tc_pre(x, W1, deg_a, deg_b)
  acc1 = _sc_scatter(g1, src, dst)
  g2 = _tc_mid(acc1[0], acc1[1], deg_a, deg_b, W2)
  acc2 = _sc_scatter(g2, src, dst)
  return _tc_post(acc2[0], acc2[1], deg_a, deg_b)


# speedup vs baseline: 3.0714x; 1.0498x over previous
import functools

import jax
import jax.numpy as jnp
from jax import lax
from jax.experimental import pallas as pl
from jax.experimental.pallas import tpu as pltpu
from jax.experimental.pallas import tpu_sc as plsc

N = 10000
E = 320000
D = 128
NC = 2
NS = 16
NW = NC * NS
CHUNK = 128
EPW = 10240
NCHUNK = EPW // CHUNK
GRP = 8
NGRP = NCHUNK // GRP
E_PAD = NW * EPW
ACC_ROWS = 10240
RPT = ACC_ROWS // NS
HROWS = ACC_ROWS // 16

_mesh = plsc.VectorSubcoreMesh(core_axis_name="c", subcore_axis_name="s")
_sc_params = pltpu.CompilerParams(needs_layout_passes=False)


@functools.partial(
    pl.kernel,
    out_type=jax.ShapeDtypeStruct((NC, HROWS, 16), jnp.float32),
    mesh=_mesh,
    compiler_params=_sc_params,
    scratch_types=[
        pltpu.VMEM((NCHUNK, CHUNK), jnp.int32),
        pltpu.VMEM((HROWS, 16), jnp.float32),
        pltpu.VMEM((HROWS // NS, 16), jnp.float32),
        pltpu.VMEM((HROWS // CHUNK, CHUNK), jnp.int32),
        pltpu.VMEM_SHARED((HROWS, 16), jnp.float32),
    ],
)
def _sc_degree(dst_hbm, deg_hbm, dstv, hist, zbuf, rowidx, deg_sh):
  cid = lax.axis_index("c")
  sid = lax.axis_index("s")
  wid = sid * NC + cid
  zero16 = jnp.zeros((16,), jnp.float32)
  rpt = HROWS // NS

  pltpu.sync_copy(dst_hbm.at[wid], dstv)
  for r in range(rpt):
    zbuf[r, :] = zero16
  pltpu.sync_copy(zbuf, deg_sh.at[pl.ds(sid * rpt, rpt), :])

  def _zero_hist(i, c):
    hist[i, :] = zero16
    return c
  lax.fori_loop(0, HROWS, _zero_hist, 0)

  for c in range(HROWS // CHUNK):
    for k in range(CHUNK // 16):
      rowidx[c, pl.ds(k * 16, 16)] = (
          c * CHUNK + k * 16 + lax.iota(jnp.int32, 16))
  plsc.subcore_barrier()

  one16 = jnp.ones((16,), jnp.float32)

  def _accum(j, c):
    for k in range(CHUNK // 16):
      d = dstv[j, pl.ds(k * 16, 16)]
      plsc.addupdate_scatter(hist, [d >> 4, d & 15], one16)
    return c
  lax.fori_loop(0, NCHUNK, _accum, 0)

  for c in range(HROWS // CHUNK):
    pltpu.sync_copy(hist.at[pl.ds(c * CHUNK, CHUNK), :],
                    deg_sh.at[rowidx.at[c]], add=True)
  plsc.subcore_barrier()

  pltpu.sync_copy(deg_sh.at[pl.ds(sid * rpt, rpt), :], zbuf)
  pltpu.sync_copy(zbuf, deg_hbm.at[cid].at[pl.ds(sid * rpt, rpt), :])


@functools.partial(
    pl.kernel,
    out_type=jax.ShapeDtypeStruct((NC, ACC_ROWS, D), jnp.float32),
    mesh=_mesh,
    compiler_params=_sc_params,
    scratch_types=[
        pltpu.VMEM((GRP, CHUNK), jnp.int32),
        pltpu.VMEM((GRP, CHUNK), jnp.int32),
        pltpu.VMEM((NCHUNK, CHUNK), jnp.int32),
        pltpu.VMEM((CHUNK, D), jnp.float32),
        pltpu.VMEM((CHUNK, D), jnp.float32),
        pltpu.VMEM_SHARED((ACC_ROWS, D), jnp.float32),
        pltpu.SemaphoreType.DMA,
        pltpu.SemaphoreType.DMA,
        pltpu.SemaphoreType.DMA,
    ],
)
def _sc_scatter(g_hbm, src_hbm, dst_hbm, out_hbm,
                srcA, srcB, dstv, buf0, buf1, acc, gsem0, gsem1, psem):
  cid = lax.axis_index("c")
  sid = lax.axis_index("s")
  wid = sid * NC + cid
  zero16 = jnp.zeros((16,), jnp.float32)

  bufs = (buf0, buf1)
  gsems = (gsem0, gsem1)
  sbufs = (srcA, srcB)

  def src_row(jj):
    return sbufs[(jj // GRP) % 2].at[jj % GRP]

  pltpu.sync_copy(dst_hbm.at[wid], dstv)
  pltpu.sync_copy(src_hbm.at[wid].at[pl.ds(0, GRP)], srcA)
  pref = pltpu.async_copy(src_hbm.at[wid].at[pl.ds(GRP, GRP)], srcB, psem)

  def _zero(i, c):
    for k in range(D // 16):
      buf0[i, pl.ds(k * 16, 16)] = zero16
    return c
  lax.fori_loop(0, CHUNK, _zero, 0)
  for c in range(RPT // CHUNK):
    pltpu.sync_copy(buf0, acc.at[pl.ds(sid * RPT + c * CHUNK, CHUNK), :])

  pend = [None, None]
  pend[0] = pltpu.async_copy(g_hbm.at[src_row(0)], buf0, gsem0)
  plsc.subcore_barrier()

  for j in range(NCHUNK):
    b = j % 2
    g = j // GRP
    if j % GRP == 0 and 0 < g and g + 1 < NGRP:
      pref = pltpu.async_copy(
          src_hbm.at[wid].at[pl.ds((g + 1) * GRP, GRP)], sbufs[(g + 1) % 2],
          psem)
    jn = j + 1
    if jn < NCHUNK:
      if jn % GRP == 0:
        pref.wait()
      pend[1 - b] = pltpu.async_copy(
          g_hbm.at[src_row(jn)], bufs[1 - b], gsems[1 - b])
    pend[b].wait()
    pltpu.sync_copy(bufs[b], acc.at[dstv.at[j]], add=True)
  plsc.subcore_barrier()

  for c in range(RPT // CHUNK):
    pltpu.sync_copy(acc.at[pl.ds(sid * RPT + c * CHUNK, CHUNK), :], buf0)
    pltpu.sync_copy(
        buf0, out_hbm.at[cid].at[pl.ds(sid * RPT + c * CHUNK, CHUNK), :])


_BLK = 1000


def _tc_pre_body(x_ref, w_ref, da_ref, db_ref, g_ref):
  deg = da_ref[...] + db_ref[...]
  dis = lax.rsqrt(deg)
  g_ref[...] = jnp.dot(x_ref[...], w_ref[...],
                       preferred_element_type=jnp.float32) * dis


def _tc_mid_body(a_ref, b_ref, da_ref, db_ref, w_ref, g_ref):
  deg = da_ref[...] + db_ref[...]
  dis = lax.rsqrt(deg)
  dis_safe = jnp.where(deg > 0, dis, 0.0)
  h = jnp.maximum((a_ref[0] + b_ref[0]) * dis_safe, 0.0)
  g_ref[...] = jnp.dot(h, w_ref[...],
                       preferred_element_type=jnp.float32) * dis


def _tc_post_body(a_ref, b_ref, da_ref, db_ref, o_ref):
  deg = da_ref[...] + db_ref[...]
  dis_safe = jnp.where(deg > 0, lax.rsqrt(deg), 0.0)
  o_ref[...] = jnp.maximum((a_ref[0] + b_ref[0]) * dis_safe, 0.0)


_row_spec = pl.BlockSpec((_BLK, D), lambda i: (i, 0))
_deg_spec = pl.BlockSpec((_BLK, 1), lambda i: (i, 0))
_w_spec = pl.BlockSpec((D, D), lambda i: (0, 0))
_out_struct = jax.ShapeDtypeStruct((N, D), jnp.float32)
_acc0_spec = pl.BlockSpec((1, _BLK, D), lambda i: (0, i, 0))
_acc1_spec = pl.BlockSpec((1, _BLK, D), lambda i: (1, i, 0))

_tc_pre = pl.pallas_call(
    _tc_pre_body,
    grid=(N // _BLK,),
    in_specs=[_row_spec, _w_spec, _deg_spec, _deg_spec],
    out_specs=_row_spec,
    out_shape=_out_struct,
)

_tc_mid = pl.pallas_call(
    _tc_mid_body,
    grid=(N // _BLK,),
    in_specs=[_acc0_spec, _acc1_spec, _deg_spec, _deg_spec, _w_spec],
    out_specs=_row_spec,
    out_shape=_out_struct,
)

_tc_post = pl.pallas_call(
    _tc_post_body,
    grid=(N // _BLK,),
    in_specs=[_acc0_spec, _acc1_spec, _deg_spec, _deg_spec],
    out_specs=_row_spec,
    out_shape=_out_struct,
)


_PAD_SRC = jnp.arange(E_PAD - E, dtype=jnp.int32) % N
_PAD_DST = N + jnp.arange(E_PAD - E, dtype=jnp.int32) % (ACC_ROWS - N)


@jax.jit
def kernel(x, edge_index, W1, W2):
  ei = edge_index.astype(jnp.int32)
  src = jnp.concatenate([ei[0], _PAD_SRC]).reshape(NW, NCHUNK, CHUNK)
  dst = jnp.concatenate([ei[1], _PAD_DST]).reshape(NW, NCHUNK, CHUNK)

  deg_parts = _sc_degree(dst)
  deg_a = deg_parts[0].reshape(ACC_ROWS, 1)
  deg_b = deg_parts[1].reshape(ACC_ROWS, 1)

  g1 = _tc_pre(x, W1, deg_a, deg_b)
  acc1 = _sc_scatter(g1, src, dst)
  g2 = _tc_mid(acc1, acc1, deg_a, deg_b, W2)
  acc2 = _sc_scatter(g2, src, dst)
  return _tc_post(acc2, acc2, deg_a, deg_b)


# speedup vs baseline: 3.1601x; 1.0289x over previous
import functools

import jax
import jax.numpy as jnp
from jax import lax
from jax.experimental import pallas as pl
from jax.experimental.pallas import tpu as pltpu
from jax.experimental.pallas import tpu_sc as plsc

N = 10000
E = 320000
D = 128
NC = 2
NS = 16
NW = NC * NS
CHUNK = 128
EPW = 10240
NCHUNK = EPW // CHUNK
GRP = 8
NGRP = NCHUNK // GRP
E_PAD = NW * EPW
ACC_ROWS = 10240
RPT = ACC_ROWS // NS
HROWS = ACC_ROWS // 16

_mesh = plsc.VectorSubcoreMesh(core_axis_name="c", subcore_axis_name="s")
_sc_params = pltpu.CompilerParams(needs_layout_passes=False)


@functools.partial(
    pl.kernel,
    out_type=jax.ShapeDtypeStruct((NC, HROWS, 16), jnp.float32),
    mesh=_mesh,
    compiler_params=_sc_params,
    scratch_types=[
        pltpu.VMEM((NCHUNK, CHUNK), jnp.int32),
        pltpu.VMEM((HROWS, 16), jnp.float32),
        pltpu.VMEM((HROWS // NS, 16), jnp.float32),
        pltpu.VMEM((HROWS // CHUNK, CHUNK), jnp.int32),
        pltpu.VMEM_SHARED((HROWS, 16), jnp.float32),
    ],
)
def _sc_degree(dst_hbm, deg_hbm, dstv, hist, zbuf, rowidx, deg_sh):
  cid = lax.axis_index("c")
  sid = lax.axis_index("s")
  wid = sid * NC + cid
  zero16 = jnp.zeros((16,), jnp.float32)
  rpt = HROWS // NS

  pltpu.sync_copy(dst_hbm.at[wid], dstv)
  for r in range(rpt):
    zbuf[r, :] = zero16
  pltpu.sync_copy(zbuf, deg_sh.at[pl.ds(sid * rpt, rpt), :])

  def _zero_hist(i, c):
    hist[i, :] = zero16
    return c
  lax.fori_loop(0, HROWS, _zero_hist, 0)

  for c in range(HROWS // CHUNK):
    for k in range(CHUNK // 16):
      rowidx[c, pl.ds(k * 16, 16)] = (
          c * CHUNK + k * 16 + lax.iota(jnp.int32, 16))
  plsc.subcore_barrier()

  one16 = jnp.ones((16,), jnp.float32)

  def _accum(j, c):
    for k in range(CHUNK // 16):
      d = dstv[j, pl.ds(k * 16, 16)]
      plsc.addupdate_scatter(hist, [d >> 4, d & 15], one16)
    return c
  lax.fori_loop(0, NCHUNK, _accum, 0)

  for c in range(HROWS // CHUNK):
    pltpu.sync_copy(hist.at[pl.ds(c * CHUNK, CHUNK), :],
                    deg_sh.at[rowidx.at[c]], add=True)
  plsc.subcore_barrier()

  pltpu.sync_copy(deg_sh.at[pl.ds(sid * rpt, rpt), :], zbuf)
  pltpu.sync_copy(zbuf, deg_hbm.at[cid].at[pl.ds(sid * rpt, rpt), :])


@functools.partial(
    pl.kernel,
    out_type=jax.ShapeDtypeStruct((NC, ACC_ROWS, D), jnp.float32),
    mesh=_mesh,
    compiler_params=_sc_params,
    scratch_types=[
        pltpu.VMEM((GRP, CHUNK), jnp.int32),
        pltpu.VMEM((GRP, CHUNK), jnp.int32),
        pltpu.VMEM((NCHUNK, CHUNK), jnp.int32),
        pltpu.VMEM((CHUNK, D), jnp.float32),
        pltpu.VMEM((CHUNK, D), jnp.float32),
        pltpu.VMEM_SHARED((ACC_ROWS, D), jnp.float32),
        pltpu.SemaphoreType.DMA,
        pltpu.SemaphoreType.DMA,
        pltpu.SemaphoreType.DMA,
    ],
)
def _sc_scatter(g_hbm, src_hbm, dst_hbm, out_hbm,
                srcA, srcB, dstv, buf0, buf1, acc, gsem0, gsem1, psem):
  cid = lax.axis_index("c")
  sid = lax.axis_index("s")
  wid = sid * NC + cid
  zero16 = jnp.zeros((16,), jnp.float32)

  bufs = (buf0, buf1)
  gsems = (gsem0, gsem1)
  sbufs = (srcA, srcB)

  def src_row(jj):
    return sbufs[(jj // GRP) % 2].at[jj % GRP]

  pltpu.sync_copy(dst_hbm.at[wid], dstv)
  pltpu.sync_copy(src_hbm.at[wid].at[pl.ds(0, GRP)], srcA)
  pref = pltpu.async_copy(src_hbm.at[wid].at[pl.ds(GRP, GRP)], srcB, psem)

  def _zero(i, c):
    for k in range(D // 16):
      buf0[i, pl.ds(k * 16, 16)] = zero16
    return c
  lax.fori_loop(0, CHUNK, _zero, 0)
  for c in range(RPT // CHUNK):
    pltpu.sync_copy(buf0, acc.at[pl.ds(sid * RPT + c * CHUNK, CHUNK), :])

  pend = [None, None]
  pend[0] = pltpu.async_copy(g_hbm.at[src_row(0)], buf0, gsem0)
  plsc.subcore_barrier()

  for j in range(NCHUNK):
    b = j % 2
    g = j // GRP
    if j % GRP == 0 and 0 < g and g + 1 < NGRP:
      pref = pltpu.async_copy(
          src_hbm.at[wid].at[pl.ds((g + 1) * GRP, GRP)], sbufs[(g + 1) % 2],
          psem)
    jn = j + 1
    if jn < NCHUNK:
      if jn % GRP == 0:
        pref.wait()
      pend[1 - b] = pltpu.async_copy(
          g_hbm.at[src_row(jn)], bufs[1 - b], gsems[1 - b])
    pend[b].wait()
    pltpu.sync_copy(bufs[b], acc.at[dstv.at[j]], add=True)
  plsc.subcore_barrier()

  for c in range(RPT // CHUNK):
    pltpu.sync_copy(acc.at[pl.ds(sid * RPT + c * CHUNK, CHUNK), :], buf0)
    pltpu.sync_copy(
        buf0, out_hbm.at[cid].at[pl.ds(sid * RPT + c * CHUNK, CHUNK), :])


_BLK = 1000


def _tc_pre_body(x_ref, w_ref, d_ref, g_ref):
  deg = d_ref[...]
  dis = lax.rsqrt(deg)
  g_ref[...] = jnp.dot(x_ref[...], w_ref[...],
                       preferred_element_type=jnp.float32) * dis


def _tc_mid_body(a_ref, b_ref, d_ref, w_ref, g_ref):
  deg = d_ref[...]
  dis = lax.rsqrt(deg)
  dis_safe = jnp.where(deg > 0, dis, 0.0)
  h = jnp.maximum((a_ref[0] + b_ref[0]) * dis_safe, 0.0)
  g_ref[...] = jnp.dot(h, w_ref[...],
                       preferred_element_type=jnp.float32) * dis


def _tc_post_body(a_ref, b_ref, d_ref, o_ref):
  deg = d_ref[...]
  dis_safe = jnp.where(deg > 0, lax.rsqrt(deg), 0.0)
  o_ref[...] = jnp.maximum((a_ref[0] + b_ref[0]) * dis_safe, 0.0)


_row_spec = pl.BlockSpec((_BLK, D), lambda i: (i, 0))
_deg_spec = pl.BlockSpec((_BLK, 1), lambda i: (i, 0))
_w_spec = pl.BlockSpec((D, D), lambda i: (0, 0))
_out_struct = jax.ShapeDtypeStruct((N, D), jnp.float32)
_acc0_spec = pl.BlockSpec((1, _BLK, D), lambda i: (0, i, 0))
_acc1_spec = pl.BlockSpec((1, _BLK, D), lambda i: (1, i, 0))

_tc_pre = pl.pallas_call(
    _tc_pre_body,
    grid=(N // _BLK,),
    in_specs=[_row_spec, _w_spec, _deg_spec],
    out_specs=_row_spec,
    out_shape=_out_struct,
)

_tc_mid = pl.pallas_call(
    _tc_mid_body,
    grid=(N // _BLK,),
    in_specs=[_acc0_spec, _acc1_spec, _deg_spec, _w_spec],
    out_specs=_row_spec,
    out_shape=_out_struct,
)

_tc_post = pl.pallas_call(
    _tc_post_body,
    grid=(N // _BLK,),
    in_specs=[_acc0_spec, _acc1_spec, _deg_spec],
    out_specs=_row_spec,
    out_shape=_out_struct,
)


_PAD_SRC = jnp.arange(E_PAD - E, dtype=jnp.int32) % N
_PAD_DST = N + jnp.arange(E_PAD - E, dtype=jnp.int32) % (ACC_ROWS - N)


@jax.jit
def kernel(x, edge_index, W1, W2):
  ei = edge_index.astype(jnp.int32)
  src = jnp.concatenate([ei[0], _PAD_SRC]).reshape(NW, NCHUNK, CHUNK)
  dst = jnp.concatenate([ei[1], _PAD_DST]).reshape(NW, NCHUNK, CHUNK)

  deg_parts = _sc_degree(dst)
  deg = (deg_parts[0] + deg_parts[1]).reshape(ACC_ROWS, 1)

  g1 = _tc_pre(x, W1, deg)
  acc1 = _sc_scatter(g1, src, dst)
  g2 = _tc_mid(acc1, acc1, deg, W2)
  acc2 = _sc_scatter(g2, src, dst)
  return _tc_post(acc2, acc2, deg)


# speedup vs baseline: 3.2152x; 1.0174x over previous
import functools

import jax
import jax.numpy as jnp
from jax import lax
from jax.experimental import pallas as pl
from jax.experimental.pallas import tpu as pltpu
from jax.experimental.pallas import tpu_sc as plsc

N = 10000
E = 320000
D = 128
NC = 2
NS = 16
NW = NC * NS
CHUNK = 128
EPW = 10240
NCHUNK = EPW // CHUNK
GRP = 16
NGRP = NCHUNK // GRP
E_PAD = NW * EPW
ACC_ROWS = 10240
RPT = ACC_ROWS // NS
HROWS = ACC_ROWS // 16

_mesh = plsc.VectorSubcoreMesh(core_axis_name="c", subcore_axis_name="s")
_sc_params = pltpu.CompilerParams(needs_layout_passes=False)


@functools.partial(
    pl.kernel,
    out_type=jax.ShapeDtypeStruct((NC, HROWS, 16), jnp.float32),
    mesh=_mesh,
    compiler_params=_sc_params,
    scratch_types=[
        pltpu.VMEM((NCHUNK, CHUNK), jnp.int32),
        pltpu.VMEM((HROWS, 16), jnp.float32),
        pltpu.VMEM((HROWS // NS, 16), jnp.float32),
        pltpu.VMEM((HROWS // CHUNK, CHUNK), jnp.int32),
        pltpu.VMEM_SHARED((HROWS, 16), jnp.float32),
    ],
)
def _sc_degree(dst_hbm, deg_hbm, dstv, hist, zbuf, rowidx, deg_sh):
  cid = lax.axis_index("c")
  sid = lax.axis_index("s")
  wid = sid * NC + cid
  zero16 = jnp.zeros((16,), jnp.float32)
  rpt = HROWS // NS

  pltpu.sync_copy(dst_hbm.at[wid], dstv)
  for r in range(rpt):
    zbuf[r, :] = zero16
  pltpu.sync_copy(zbuf, deg_sh.at[pl.ds(sid * rpt, rpt), :])

  def _zero_hist(i, c):
    hist[i, :] = zero16
    return c
  lax.fori_loop(0, HROWS, _zero_hist, 0)

  for c in range(HROWS // CHUNK):
    for k in range(CHUNK // 16):
      rowidx[c, pl.ds(k * 16, 16)] = (
          c * CHUNK + k * 16 + lax.iota(jnp.int32, 16))
  plsc.subcore_barrier()

  one16 = jnp.ones((16,), jnp.float32)

  def _accum(j, c):
    for k in range(CHUNK // 16):
      d = dstv[j, pl.ds(k * 16, 16)]
      plsc.addupdate_scatter(hist, [d >> 4, d & 15], one16)
    return c
  lax.fori_loop(0, NCHUNK, _accum, 0)

  for c in range(HROWS // CHUNK):
    pltpu.sync_copy(hist.at[pl.ds(c * CHUNK, CHUNK), :],
                    deg_sh.at[rowidx.at[c]], add=True)
  plsc.subcore_barrier()

  pltpu.sync_copy(deg_sh.at[pl.ds(sid * rpt, rpt), :], zbuf)
  pltpu.sync_copy(zbuf, deg_hbm.at[cid].at[pl.ds(sid * rpt, rpt), :])


@functools.partial(
    pl.kernel,
    out_type=jax.ShapeDtypeStruct((NC, ACC_ROWS, D), jnp.float32),
    mesh=_mesh,
    compiler_params=_sc_params,
    scratch_types=[
        pltpu.VMEM((GRP, CHUNK), jnp.int32),
        pltpu.VMEM((GRP, CHUNK), jnp.int32),
        pltpu.VMEM((NCHUNK, CHUNK), jnp.int32),
        pltpu.VMEM((CHUNK, D), jnp.float32),
        pltpu.VMEM((CHUNK, D), jnp.float32),
        pltpu.VMEM_SHARED((ACC_ROWS, D), jnp.float32),
        pltpu.SemaphoreType.DMA,
        pltpu.SemaphoreType.DMA,
        pltpu.SemaphoreType.DMA,
    ],
)
def _sc_scatter(g_hbm, src_hbm, dst_hbm, out_hbm,
                srcA, srcB, dstv, buf0, buf1, acc, gsem0, gsem1, psem):
  cid = lax.axis_index("c")
  sid = lax.axis_index("s")
  wid = sid * NC + cid
  zero16 = jnp.zeros((16,), jnp.float32)

  bufs = (buf0, buf1)
  gsems = (gsem0, gsem1)
  sbufs = (srcA, srcB)

  def src_row(jj):
    return sbufs[(jj // GRP) % 2].at[jj % GRP]

  pltpu.sync_copy(dst_hbm.at[wid], dstv)
  pltpu.sync_copy(src_hbm.at[wid].at[pl.ds(0, GRP)], srcA)
  pref = pltpu.async_copy(src_hbm.at[wid].at[pl.ds(GRP, GRP)], srcB, psem)

  def _zero(i, c):
    for k in range(D // 16):
      buf0[i, pl.ds(k * 16, 16)] = zero16
    return c
  lax.fori_loop(0, CHUNK, _zero, 0)
  for c in range(RPT // CHUNK):
    pltpu.sync_copy(buf0, acc.at[pl.ds(sid * RPT + c * CHUNK, CHUNK), :])

  pend = [None, None]
  pend[0] = pltpu.async_copy(g_hbm.at[src_row(0)], buf0, gsem0)
  plsc.subcore_barrier()

  for j in range(NCHUNK):
    b = j % 2
    g = j // GRP
    if j % GRP == 0 and 0 < g and g + 1 < NGRP:
      pref = pltpu.async_copy(
          src_hbm.at[wid].at[pl.ds((g + 1) * GRP, GRP)], sbufs[(g + 1) % 2],
          psem)
    jn = j + 1
    if jn < NCHUNK:
      if jn % GRP == 0:
        pref.wait()
      pend[1 - b] = pltpu.async_copy(
          g_hbm.at[src_row(jn)], bufs[1 - b], gsems[1 - b])
    pend[b].wait()
    pltpu.sync_copy(bufs[b], acc.at[dstv.at[j]], add=True)
  plsc.subcore_barrier()

  for c in range(RPT // CHUNK):
    pltpu.sync_copy(acc.at[pl.ds(sid * RPT + c * CHUNK, CHUNK), :], buf0)
    pltpu.sync_copy(
        buf0, out_hbm.at[cid].at[pl.ds(sid * RPT + c * CHUNK, CHUNK), :])


_BLK = 2000


def _tc_pre_body(x_ref, w_ref, d_ref, g_ref):
  deg = d_ref[...]
  dis = lax.rsqrt(deg)
  g_ref[...] = jnp.dot(x_ref[...], w_ref[...],
                       preferred_element_type=jnp.float32) * dis


def _tc_mid_body(a_ref, b_ref, d_ref, w_ref, g_ref):
  deg = d_ref[...]
  dis = lax.rsqrt(deg)
  dis_safe = jnp.where(deg > 0, dis, 0.0)
  h = jnp.maximum((a_ref[0] + b_ref[0]) * dis_safe, 0.0)
  g_ref[...] = jnp.dot(h, w_ref[...],
                       preferred_element_type=jnp.float32) * dis


def _tc_post_body(a_ref, b_ref, d_ref, o_ref):
  deg = d_ref[...]
  dis_safe = jnp.where(deg > 0, lax.rsqrt(deg), 0.0)
  o_ref[...] = jnp.maximum((a_ref[0] + b_ref[0]) * dis_safe, 0.0)


_row_spec = pl.BlockSpec((_BLK, D), lambda i: (i, 0))
_deg_spec = pl.BlockSpec((_BLK, 1), lambda i: (i, 0))
_w_spec = pl.BlockSpec((D, D), lambda i: (0, 0))
_out_struct = jax.ShapeDtypeStruct((N, D), jnp.float32)
_acc0_spec = pl.BlockSpec((1, _BLK, D), lambda i: (0, i, 0))
_acc1_spec = pl.BlockSpec((1, _BLK, D), lambda i: (1, i, 0))

_tc_pre = pl.pallas_call(
    _tc_pre_body,
    grid=(N // _BLK,),
    in_specs=[_row_spec, _w_spec, _deg_spec],
    out_specs=_row_spec,
    out_shape=_out_struct,
)

_tc_mid = pl.pallas_call(
    _tc_mid_body,
    grid=(N // _BLK,),
    in_specs=[_acc0_spec, _acc1_spec, _deg_spec, _w_spec],
    out_specs=_row_spec,
    out_shape=_out_struct,
)

_tc_post = pl.pallas_call(
    _tc_post_body,
    grid=(N // _BLK,),
    in_specs=[_acc0_spec, _acc1_spec, _deg_spec],
    out_specs=_row_spec,
    out_shape=_out_struct,
)


_PAD_SRC = jnp.arange(E_PAD - E, dtype=jnp.int32) % N
_PAD_DST = N + jnp.arange(E_PAD - E, dtype=jnp.int32) % (ACC_ROWS - N)


@jax.jit
def kernel(x, edge_index, W1, W2):
  ei = edge_index.astype(jnp.int32)
  src = jnp.concatenate([ei[0], _PAD_SRC]).reshape(NW, NCHUNK, CHUNK)
  dst = jnp.concatenate([ei[1], _PAD_DST]).reshape(NW, NCHUNK, CHUNK)

  deg_parts = _sc_degree(dst)
  deg = (deg_parts[0] + deg_parts[1]).reshape(ACC_ROWS, 1)

  g1 = _tc_pre(x, W1, deg)
  acc1 = _sc_scatter(g1, src, dst)
  g2 = _tc_mid(acc1, acc1, deg, W2)
  acc2 = _sc_scatter(g2, src, dst)
  return _tc_post(acc2, acc2, deg)
